# baseline (speedup 1.0000x reference)
import jax
import jax.numpy as jnp
from jax import lax
from jax.experimental import pallas as pl
from jax.experimental.pallas import tpu as pltpu
from jax.experimental.pallas import tpu_sc as plsc

_VOCAB = 1000000
_EMB = 64
_HID = 64
_PADW = 128
_B = 4096
_L = 200


_MM_BLK = 32768


def _mm_body(tT_ref, w_ref, o_ref):
    y = lax.dot_general(
        w_ref[...], tT_ref[...], (((1,), (0,)), ((), ())),
        preferred_element_type=jnp.float32)
    o_ref[:, :_HID] = y.T


def _project(tableT, W):
    return pl.pallas_call(
        _mm_body,
        grid=(pl.cdiv(_VOCAB, _MM_BLK),),
        in_specs=[
            pl.BlockSpec((_EMB, _MM_BLK), lambda i: (0, i)),
            pl.BlockSpec((_HID, _EMB), lambda i: (0, 0)),
        ],
        out_specs=pl.BlockSpec((_MM_BLK, _PADW), lambda i: (i, 0)),
        out_shape=jax.ShapeDtypeStruct((_VOCAB, _PADW), jnp.float32),
    )(tableT, W)


_NC, _NS = 2, 16
_NW = _NC * _NS
_BATCH_PER_W = _B // _NW
_NBUF = 4
_RWAVES = _BATCH_PER_W // _NBUF
_SPLITS = ((0, 128), (128, 72))


def _fire(proj, idx_v, rows_v, sem):
    return [
        pltpu.async_copy(
            proj.at[idx_v.at[pl.ds(off, ln)]],
            rows_v.at[pl.ds(off, ln)], sem)
        for off, ln in _SPLITS
    ]


def _drain_store(out_hbm, b, rows_v, copies):
    for c in copies:
        c.wait()
    pltpu.sync_copy(rows_v, out_hbm.at[b])


def _gather_body(proj, idx_hbm, out_hbm, idx_v, rows_v, sems, isems):
    wid = lax.axis_index("s") * _NC + lax.axis_index("c")
    bbase = wid * _BATCH_PER_W

    def body(w, carry):
        b0 = bbase + w * _NBUF
        icps = [
            pltpu.async_copy(idx_hbm.at[b0 + k], idx_v.at[k], isems.at[k])
            for k in range(_NBUF)
        ]
        cps = []
        for k in range(_NBUF):
            icps[k].wait()
            cps.append(_fire(proj, idx_v.at[k], rows_v.at[k], sems.at[k]))
        for k in range(_NBUF):
            _drain_store(out_hbm, b0 + k, rows_v.at[k], cps[k])
        return carry
    lax.fori_loop(0, _RWAVES, body, 0)


_gather = pl.kernel(
    _gather_body,
    out_type=jax.ShapeDtypeStruct((_B, _L, _PADW), jnp.float32),
    mesh=plsc.VectorSubcoreMesh(core_axis_name="c", subcore_axis_name="s"),
    scratch_types=[
        pltpu.VMEM((_NBUF, _L), jnp.int32),
        pltpu.VMEM((_NBUF, _L, _PADW), jnp.float32),
        pltpu.SemaphoreType.DMA((_NBUF,)),
        pltpu.SemaphoreType.DMA((_NBUF,)),
    ],
    compiler_params=pltpu.CompilerParams(use_tc_tiling_on_sc=False),
)


def kernel(prem, hypo, table, W):
    proj = _project(table.T, W)
    gp = _gather(proj, prem.astype(jnp.int32))
    gh = _gather(proj, hypo.astype(jnp.int32))
    return gp[:, :, :_HID], gh[:, :, :_HID]


# speedup vs baseline: 1.0287x; 1.0287x over previous
import jax
import jax.numpy as jnp
from jax import lax
from jax.experimental import pallas as pl
from jax.experimental.pallas import tpu as pltpu
from jax.experimental.pallas import tpu_sc as plsc

_VOCAB = 1000000
_EMB = 64
_HID = 64
_PADW = 128
_B = 4096
_L = 200


_MM_BLK = 32768


def _mm_body(tT_ref, w_ref, o_ref):
    y = lax.dot_general(
        w_ref[...], tT_ref[...], (((1,), (0,)), ((), ())),
        preferred_element_type=jnp.float32)
    o_ref[:, :_HID] = y.T


def _project(tableT, W):
    return pl.pallas_call(
        _mm_body,
        grid=(pl.cdiv(_VOCAB, _MM_BLK),),
        in_specs=[
            pl.BlockSpec((_EMB, _MM_BLK), lambda i: (0, i)),
            pl.BlockSpec((_HID, _EMB), lambda i: (0, 0)),
        ],
        out_specs=pl.BlockSpec((_MM_BLK, _PADW), lambda i: (i, 0)),
        out_shape=jax.ShapeDtypeStruct((_VOCAB, _PADW), jnp.float32),
    )(tableT, W)


_NC, _NS = 2, 16
_NW = _NC * _NS
_BATCH_PER_W = _B // _NW
_NBUF = 4
_RWAVES = _BATCH_PER_W // _NBUF
_SPLITS = ((0, 128), (128, 72))


def _fire(proj, idx_hbm, b, idx_v, rows_v, sem):
    pltpu.sync_copy(idx_hbm.at[b], idx_v)
    return [
        pltpu.async_copy(
            proj.at[idx_v.at[pl.ds(off, ln)]],
            rows_v.at[pl.ds(off, ln)], sem)
        for off, ln in _SPLITS
    ]


def _drain_store(out_hbm, b, rows_v, copies):
    for c in copies:
        c.wait()
    pltpu.sync_copy(rows_v, out_hbm.at[b])


def _gather_body(proj, idx_hbm, out_hbm, idx_v, rows_v, sems):
    wid = lax.axis_index("s") * _NC + lax.axis_index("c")
    bbase = wid * _BATCH_PER_W

    def body(w, carry):
        b0 = bbase + w * _NBUF
        cps = [
            _fire(proj, idx_hbm, b0 + k, idx_v.at[k], rows_v.at[k],
                  sems.at[k])
            for k in range(_NBUF)
        ]
        for k in range(_NBUF):
            _drain_store(out_hbm, b0 + k, rows_v.at[k], cps[k])
        return carry
    lax.fori_loop(0, _RWAVES, body, 0)


_gather = pl.kernel(
    _gather_body,
    out_type=jax.ShapeDtypeStruct((_B, _L, _PADW), jnp.float32),
    mesh=plsc.VectorSubcoreMesh(core_axis_name="c", subcore_axis_name="s"),
    scratch_types=[
        pltpu.VMEM((_NBUF, _L), jnp.int32),
        pltpu.VMEM((_NBUF, _L, _PADW), jnp.float32),
        pltpu.SemaphoreType.DMA((_NBUF,)),
    ],
    compiler_params=pltpu.CompilerParams(use_tc_tiling_on_sc=False),
)


def kernel(prem, hypo, table, W):
    proj = _project(table.T, W)
    gp = _gather(proj, prem.astype(jnp.int32))
    gh = _gather(proj, hypo.astype(jnp.int32))
    return gp[:, :, :_HID], gh[:, :, :_HID]
